# baseline (device time: 206320 ns/iter reference)
import jax
import jax.numpy as jnp
from jax import lax
from jax.experimental import pallas as pl
from jax.experimental.pallas import tpu as pltpu

N_DEV = 8


def kernel(x, w_mat):
    K, k_per = x.shape
    _, N = w_mat.shape
    m_per = K // N_DEV
    NB = 2
    n_blk = N // NB
    KH = 2
    khw = k_per // KH

    def body(x_ref, w_ref, out_ref, comm_ref, sendbuf_ref, stage_ref,
             wbuf_ref, send_sems, recv_sems, stage_sems, w_sems):
        i = lax.axis_index("i")

        q = lax.rem(i, 4)
        zb = i // 4
        qx = jnp.bitwise_xor(q, 1)
        qy = 3 - q
        qxy = lax.rem(q + 2, 4)
        zs = 4 * zb
        zo = 4 * (1 - zb)
        peers = [qx + zs, qy + zs, q + zo,
                 qxy + zs, qx + zo, qy + zo,
                 qxy + zo]

        def start_w(t, kh, nb, slot):
            src = i if t == 0 else peers[t - 1]
            cp = pltpu.make_async_copy(
                w_ref.at[pl.ds(src * k_per + kh * khw, khw),
                         pl.ds(nb * n_blk, n_blk)],
                wbuf_ref.at[slot],
                w_sems.at[slot],
            )
            cp.start()
            return cp

        steps = [(t, kh, nb)
                 for t in range(N_DEV) for kh in range(KH) for nb in range(NB)]
        pend_w = {0: start_w(*steps[0], 0), 1: start_w(*steps[1], 1)}

        cast_order = [7, 6, 5, 4, 1, 2, 3, 0]
        jobs = [(t, kh) for t in cast_order for kh in range(KH)]

        def stage_start(idx, slot):
            t, kh = jobs[idx]
            j = i if t == 0 else peers[t - 1]
            cp = pltpu.make_async_copy(
                x_ref.at[pl.ds(j * m_per, m_per), pl.ds(kh * khw, khw)],
                stage_ref.at[slot],
                stage_sems.at[slot],
            )
            cp.start()
            return cp

        pend_stage = {0: stage_start(0, 0), 1: stage_start(1, 1)}
        rdmas = {}
        for idx, (t, kh) in enumerate(jobs):
            slot = idx % 2
            pend_stage[slot].wait()
            half_bf = stage_ref[slot].astype(jnp.bfloat16)
            if t == 0:
                comm_ref[0, kh] = half_bf
            else:
                sendbuf_ref[t - 1, kh] = half_bf
            if idx + 2 < len(jobs):
                pend_stage[slot] = stage_start(idx + 2, slot)
            if t > 0:
                rdma = pltpu.make_async_remote_copy(
                    src_ref=sendbuf_ref.at[t - 1, kh],
                    dst_ref=comm_ref.at[t, kh],
                    send_sem=send_sems.at[KH * t + kh],
                    recv_sem=recv_sems.at[KH * t + kh],
                    device_id=(peers[t - 1],),
                    device_id_type=pl.DeviceIdType.MESH,
                )
                rdma.start()
                rdmas[(t, kh)] = rdma

        for idx, (t, kh, nb) in enumerate(steps):
            slot = idx % 2
            pend_w[slot].wait()
            if nb == 0 and t > 0:
                rdmas[(t, kh)].wait_recv()
            part = lax.dot_general(
                comm_ref[t, kh],
                wbuf_ref[slot].astype(jnp.bfloat16),
                dimension_numbers=(((1,), (0,)), ((), ())),
                preferred_element_type=jnp.float32,
            )
            ocols = pl.ds(nb * n_blk, n_blk)
            if t == 0 and kh == 0:
                out_ref[:, ocols] = part
            else:
                out_ref[:, ocols] += part
            if idx + 2 < len(steps):
                pend_w[slot] = start_w(*steps[idx + 2], slot)

        for rdma in rdmas.values():
            rdma.wait_send()

    return pl.pallas_call(
        body,
        out_shape=jax.ShapeDtypeStruct((m_per, N), jnp.float32),
        in_specs=[
            pl.BlockSpec(memory_space=pl.ANY),
            pl.BlockSpec(memory_space=pl.ANY),
        ],
        out_specs=pl.BlockSpec(memory_space=pltpu.VMEM),
        scratch_shapes=[
            pltpu.VMEM((N_DEV, KH, m_per, khw), jnp.bfloat16),
            pltpu.VMEM((N_DEV - 1, KH, m_per, khw), jnp.bfloat16),
            pltpu.VMEM((2, m_per, khw), jnp.float32),
            pltpu.VMEM((2, khw, n_blk), jnp.float32),
            pltpu.SemaphoreType.DMA((N_DEV * KH,)),
            pltpu.SemaphoreType.DMA((N_DEV * KH,)),
            pltpu.SemaphoreType.DMA((2,)),
            pltpu.SemaphoreType.DMA((2,)),
        ],
        compiler_params=pltpu.CompilerParams(
            vmem_limit_bytes=63 * 1024 * 1024,
        ),
    )(x, w_mat)


# device time: 167695 ns/iter; 1.2303x vs baseline; 1.2303x over previous
import jax
import jax.numpy as jnp
from jax import lax
from jax.experimental import pallas as pl
from jax.experimental.pallas import tpu as pltpu

N_DEV = 8


def kernel(x, w_mat):
    K, k_per = x.shape
    _, N = w_mat.shape
    m_per = K // N_DEV
    NB = 2
    n_blk = N // NB
    KH = 2
    khw = k_per // KH

    def body(x_ref, w_ref, out_ref, comm_ref, sendbuf_ref, stage_ref,
             wbuf_ref, send_sems, recv_sems, stage_sems, w_sems):
        i = lax.axis_index("i")

        q = lax.rem(i, 4)
        zb = i // 4
        qx = jnp.bitwise_xor(q, 1)
        qy = 3 - q
        qxy = lax.rem(q + 2, 4)
        zs = 4 * zb
        zo = 4 * (1 - zb)
        peers = [qx + zs, qy + zs, q + zo,
                 qxy + zs, qx + zo, qy + zo,
                 qxy + zo]

        def start_w(t, kh, nb, slot):
            src = i if t == 0 else peers[t - 1]
            cp = pltpu.make_async_copy(
                w_ref.at[pl.ds(src * k_per + kh * khw, khw),
                         pl.ds(nb * n_blk, n_blk)],
                wbuf_ref.at[slot],
                w_sems.at[slot],
            )
            cp.start()
            return cp

        steps = [(t, kh, nb)
                 for t in range(N_DEV) for kh in range(KH) for nb in range(NB)]
        pend_w = {0: start_w(*steps[0], 0), 1: start_w(*steps[1], 1)}

        cast_order = [7, 1, 2, 3, 4, 5, 6, 0]
        jobs = [(t, kh) for t in cast_order for kh in range(KH)]

        def stage_start(idx, slot):
            t, kh = jobs[idx]
            j = i if t == 0 else peers[t - 1]
            cp = pltpu.make_async_copy(
                x_ref.at[pl.ds(j * m_per, m_per), pl.ds(kh * khw, khw)],
                stage_ref.at[slot],
                stage_sems.at[slot],
            )
            cp.start()
            return cp

        pend_stage = {0: stage_start(0, 0), 1: stage_start(1, 1)}
        rdmas = {}
        for idx, (t, kh) in enumerate(jobs):
            slot = idx % 2
            pend_stage[slot].wait()
            half_bf = stage_ref[slot].astype(jnp.bfloat16)
            if t == 0:
                comm_ref[0, kh] = half_bf
            else:
                sendbuf_ref[t - 1, kh] = half_bf
            if idx + 2 < len(jobs):
                pend_stage[slot] = stage_start(idx + 2, slot)
            if t > 0:
                rdma = pltpu.make_async_remote_copy(
                    src_ref=sendbuf_ref.at[t - 1, kh],
                    dst_ref=comm_ref.at[t, kh],
                    send_sem=send_sems.at[KH * t + kh],
                    recv_sem=recv_sems.at[KH * t + kh],
                    device_id=(peers[t - 1],),
                    device_id_type=pl.DeviceIdType.MESH,
                )
                rdma.start()
                rdmas[(t, kh)] = rdma

        for idx, (t, kh, nb) in enumerate(steps):
            slot = idx % 2
            pend_w[slot].wait()
            if nb == 0 and t > 0:
                rdmas[(t, kh)].wait_recv()
            part = lax.dot_general(
                comm_ref[t, kh],
                wbuf_ref[slot].astype(jnp.bfloat16),
                dimension_numbers=(((1,), (0,)), ((), ())),
                preferred_element_type=jnp.float32,
            )
            ocols = pl.ds(nb * n_blk, n_blk)
            if t == 0 and kh == 0:
                out_ref[:, ocols] = part
            else:
                out_ref[:, ocols] += part
            if idx + 2 < len(steps):
                pend_w[slot] = start_w(*steps[idx + 2], slot)

        for rdma in rdmas.values():
            rdma.wait_send()

    return pl.pallas_call(
        body,
        out_shape=jax.ShapeDtypeStruct((m_per, N), jnp.float32),
        in_specs=[
            pl.BlockSpec(memory_space=pl.ANY),
            pl.BlockSpec(memory_space=pl.ANY),
        ],
        out_specs=pl.BlockSpec(memory_space=pltpu.VMEM),
        scratch_shapes=[
            pltpu.VMEM((N_DEV, KH, m_per, khw), jnp.bfloat16),
            pltpu.VMEM((N_DEV - 1, KH, m_per, khw), jnp.bfloat16),
            pltpu.VMEM((2, m_per, khw), jnp.float32),
            pltpu.VMEM((2, khw, n_blk), jnp.float32),
            pltpu.SemaphoreType.DMA((N_DEV * KH,)),
            pltpu.SemaphoreType.DMA((N_DEV * KH,)),
            pltpu.SemaphoreType.DMA((2,)),
            pltpu.SemaphoreType.DMA((2,)),
        ],
        compiler_params=pltpu.CompilerParams(
            vmem_limit_bytes=63 * 1024 * 1024,
        ),
    )(x, w_mat)
